# baseline (speedup 1.0000x reference)
import functools

import jax
import jax.numpy as jnp
from jax import lax
from jax.experimental import pallas as pl
from jax.experimental.pallas import tpu as pltpu
from jax.experimental.pallas import tpu_sc as plsc

NC = 2
NS = 16
NW = NC * NS
L = 16

D = 64
DC = D // L
IB = 4
G_PAD = 96


def _sc_body(K, NBLK,
             w_hbm, g_hbm, accp_hbm, accn_hbm,
             g_v, rows, accp_v, accn_v, sem):
  KA = K + 2
  GA = KA * IB
  w = lax.axis_index("s") * NC + lax.axis_index("c")
  ipw = NBLK * IB

  pltpu.sync_copy(g_hbm.at[w], g_v)

  def gather_descr(blk, par):
    goff = pl.multiple_of(blk * G_PAD, 8)
    return pltpu.make_async_copy(w_hbm.at[g_v.at[pl.ds(goff, GA)]],
                                 rows.at[par], sem.at[par])

  gather_descr(0, 0).start()

  def half(g, _):
    for par in range(2):
      blk = g * 2 + par
      nxt = blk + 1

      @pl.when(nxt < NBLK)
      def _():
        gather_descr(nxt, (par + 1) % 2).start()

      gather_descr(blk, par).wait()

      for i in range(IB):
        a = i * KA
        vc = [rows[par, a, pl.ds(c * L, L)] for c in range(DC)]

        def row_fma(r, acc4):
          return [acc4[c] + rows[par, r, pl.ds(D + c * L, L)] * vc[c]
                  for c in range(DC)]

        accp4 = row_fma(a + 1, [jnp.zeros((L,), jnp.float32)] * DC)
        accn4 = [jnp.zeros((L,), jnp.float32)] * DC
        for k in range(K):
          accn4 = row_fma(a + 2 + k, accn4)
        ooff = pl.multiple_of((blk * IB + i) * L, 8)
        accp_v[pl.ds(ooff, L)] = (accp4[0] + accp4[1]) + (accp4[2] + accp4[3])
        accn_v[pl.ds(ooff, L)] = (accn4[0] + accn4[1]) + (accn4[2] + accn4[3])
    return 0

  lax.fori_loop(0, NBLK // 2, half, 0)

  obase = pl.multiple_of(w * ipw * L, 8)
  pltpu.sync_copy(accp_v, accp_hbm.at[pl.ds(obase, ipw * L)])
  pltpu.sync_copy(accn_v, accn_hbm.at[pl.ds(obase, ipw * L)])


def _tc_buildw(v_ref, u_ref, w_ref):
  w_ref[...] = jnp.concatenate([v_ref[...], u_ref[...]], axis=1)


def _tc_finish(nitems, accp_ref, accn_ref, out_ref):
  rows, lanes = accp_ref.shape
  g = lanes // L
  sel = (lax.broadcasted_iota(jnp.int32, (lanes, g), 0) // L ==
         lax.broadcasted_iota(jnp.int32, (lanes, g), 1)).astype(jnp.float32)
  pos = jnp.dot(accp_ref[...], sel, preferred_element_type=jnp.float32)
  negdot = jnp.dot(accn_ref[...], sel, preferred_element_type=jnp.float32)
  loss = jax.nn.log_sigmoid(pos) + jax.nn.log_sigmoid(-negdot)
  out_ref[0, 0] = -jnp.sum(loss) / nitems


def kernel(V, U, center_words, target_words, neg_words):
  B, K = neg_words.shape
  KA = K + 2
  GA = KA * IB
  ipw = B // NW
  NBLK = ipw // IB

  VOC = V.shape[0]
  R = 8000
  W = pl.pallas_call(
      _tc_buildw,
      grid=(VOC // R,),
      in_specs=[pl.BlockSpec((R, D), lambda i: (i, 0)),
                pl.BlockSpec((R, D), lambda i: (i, 0))],
      out_specs=pl.BlockSpec((R, 2 * D), lambda i: (i, 0)),
      out_shape=jax.ShapeDtypeStruct((VOC, 2 * D), jnp.float32),
  )(V, U)

  gidx = jnp.concatenate([center_words, target_words, neg_words], axis=1)
  gidx = gidx.astype(jnp.int32).reshape(NW, NBLK, GA)
  gidx = jnp.pad(gidx, ((0, 0), (0, 0), (0, G_PAD - GA)))

  sc = pl.kernel(
      functools.partial(_sc_body, K, NBLK),
      out_type=(jax.ShapeDtypeStruct((B * L,), jnp.float32),
                jax.ShapeDtypeStruct((B * L,), jnp.float32)),
      mesh=plsc.VectorSubcoreMesh(core_axis_name="c", subcore_axis_name="s"),
      compiler_params=pltpu.CompilerParams(use_tc_tiling_on_sc=True),
      scratch_types=[
          pltpu.VMEM((NBLK * G_PAD,), jnp.int32),
          pltpu.VMEM((2, GA, 2 * D), jnp.float32),
          pltpu.VMEM((ipw * L,), jnp.float32),
          pltpu.VMEM((ipw * L,), jnp.float32),
          pltpu.SemaphoreType.DMA((2,)),
      ],
  )
  accp, accn = sc(W, gidx.reshape(NW, NBLK * G_PAD))

  out = pl.pallas_call(
      functools.partial(_tc_finish, float(B)),
      out_shape=jax.ShapeDtypeStruct((1, 1), jnp.float32),
      out_specs=pl.BlockSpec(memory_space=pltpu.SMEM),
  )(accp.reshape(B * L // 128, 128), accn.reshape(B * L // 128, 128))
  return out.reshape(())


# speedup vs baseline: 1.4112x; 1.4112x over previous
import functools

import jax
import jax.numpy as jnp
from jax import lax
from jax.experimental import pallas as pl
from jax.experimental.pallas import tpu as pltpu
from jax.experimental.pallas import tpu_sc as plsc

NC = 2
NS = 16
NW = NC * NS
L = 16

D = 64
DC = D // L
IB = 4
G_PAD = 96


def _sc_body(K, NBLK,
             v_hbm, u_hbm, g_hbm, accp_hbm, accn_hbm,
             g_v, rows, accp_v, accn_v, sem):
  KA = K + 2
  GA = KA * IB
  w = lax.axis_index("s") * NC + lax.axis_index("c")
  ipw = NBLK * IB

  pltpu.sync_copy(g_hbm.at[w], g_v)

  def src_tab(j):
    return v_hbm if j % KA == 0 else u_hbm

  def gather_start(blk, par):
    goff = pl.multiple_of(blk * G_PAD, 8)
    idxv = [g_v[pl.ds(goff + c * L, L)] for c in range(G_PAD // L)]
    for j in range(GA):
      idx = idxv[j // L][j % L]
      pltpu.async_copy(src_tab(j).at[pl.ds(idx, 1)],
                       rows.at[par].at[pl.ds(j, 1)], sem.at[par])

  def gather_wait(blk, par):
    for j in range(GA):
      pltpu.make_async_copy(src_tab(j).at[pl.ds(0, 1)],
                            rows.at[par].at[pl.ds(j, 1)],
                            sem.at[par]).wait()

  gather_start(0, 0)

  def half(g, _):
    for par in range(2):
      blk = g * 2 + par
      nxt = blk + 1

      @pl.when(nxt < NBLK)
      def _():
        gather_start(nxt, (par + 1) % 2)

      gather_wait(blk, par)

      for i in range(IB):
        a = i * KA
        vc = [rows[par, a, pl.ds(c * L, L)] for c in range(DC)]

        def row_fma(r, acc4):
          return [acc4[c] + rows[par, r, pl.ds(c * L, L)] * vc[c]
                  for c in range(DC)]

        accp4 = row_fma(a + 1, [jnp.zeros((L,), jnp.float32)] * DC)
        accn4 = [jnp.zeros((L,), jnp.float32)] * DC
        for k in range(K):
          accn4 = row_fma(a + 2 + k, accn4)
        ooff = pl.multiple_of((blk * IB + i) * L, 8)
        accp_v[pl.ds(ooff, L)] = (accp4[0] + accp4[1]) + (accp4[2] + accp4[3])
        accn_v[pl.ds(ooff, L)] = (accn4[0] + accn4[1]) + (accn4[2] + accn4[3])
    return 0

  lax.fori_loop(0, NBLK // 2, half, 0)

  obase = pl.multiple_of(w * ipw * L, 8)
  pltpu.sync_copy(accp_v, accp_hbm.at[pl.ds(obase, ipw * L)])
  pltpu.sync_copy(accn_v, accn_hbm.at[pl.ds(obase, ipw * L)])


def _tc_finish(nitems, accp_ref, accn_ref, out_ref):
  rows, lanes = accp_ref.shape
  g = lanes // L
  sel = (lax.broadcasted_iota(jnp.int32, (lanes, g), 0) // L ==
         lax.broadcasted_iota(jnp.int32, (lanes, g), 1)).astype(jnp.float32)
  pos = jnp.dot(accp_ref[...], sel, preferred_element_type=jnp.float32)
  negdot = jnp.dot(accn_ref[...], sel, preferred_element_type=jnp.float32)
  loss = jax.nn.log_sigmoid(pos) + jax.nn.log_sigmoid(-negdot)
  out_ref[0, 0] = -jnp.sum(loss) / nitems


def kernel(V, U, center_words, target_words, neg_words):
  B, K = neg_words.shape
  KA = K + 2
  GA = KA * IB
  ipw = B // NW
  NBLK = ipw // IB

  gidx = jnp.concatenate([center_words, target_words, neg_words], axis=1)
  gidx = gidx.astype(jnp.int32).reshape(NW, NBLK, GA)
  gidx = jnp.pad(gidx, ((0, 0), (0, 0), (0, G_PAD - GA)))

  sc = pl.kernel(
      functools.partial(_sc_body, K, NBLK),
      out_type=(jax.ShapeDtypeStruct((B * L,), jnp.float32),
                jax.ShapeDtypeStruct((B * L,), jnp.float32)),
      mesh=plsc.VectorSubcoreMesh(core_axis_name="c", subcore_axis_name="s"),
      compiler_params=pltpu.CompilerParams(use_tc_tiling_on_sc=True),
      scratch_types=[
          pltpu.VMEM((NBLK * G_PAD,), jnp.int32),
          pltpu.VMEM((2, GA, D), jnp.float32),
          pltpu.VMEM((ipw * L,), jnp.float32),
          pltpu.VMEM((ipw * L,), jnp.float32),
          pltpu.SemaphoreType.DMA((2,)),
      ],
  )
  accp, accn = sc(V, U, gidx.reshape(NW, NBLK * G_PAD))

  out = pl.pallas_call(
      functools.partial(_tc_finish, float(B)),
      out_shape=jax.ShapeDtypeStruct((1, 1), jnp.float32),
      out_specs=pl.BlockSpec(memory_space=pltpu.SMEM),
  )(accp.reshape(B * L // 128, 128), accn.reshape(B * L // 128, 128))
  return out.reshape(())


# speedup vs baseline: 1.6633x; 1.1787x over previous
import functools

import jax
import jax.numpy as jnp
from jax import lax
from jax.experimental import pallas as pl
from jax.experimental.pallas import tpu as pltpu
from jax.experimental.pallas import tpu_sc as plsc

NC = 2
NS = 16
NW = NC * NS
L = 16

D = 64
DC = D // L
IB = 4
G_PAD = 96


def _sc_body(K, NBLK,
             v_hbm, u_hbm, g_hbm, accp_hbm, accn_hbm,
             g_v, rows, accp_v, accn_v, sem):
  KA = K + 2
  GA = KA * IB
  w = lax.axis_index("s") * NC + lax.axis_index("c")
  ipw = NBLK * IB

  pltpu.sync_copy(g_hbm.at[w], g_v)

  def src_tab(j):
    return v_hbm if j % KA == 0 else u_hbm

  NQ = 4

  def gather_start(blk, par):
    goff = pl.multiple_of(blk * G_PAD, 8)
    idxv = [g_v[pl.ds(goff + c * L, L)] for c in range(G_PAD // L)]
    for j in range(GA):
      idx = idxv[j // L][j % L]
      pltpu.async_copy(src_tab(j).at[pl.ds(idx, 1)],
                       rows.at[par].at[pl.ds(j, 1)], sem.at[par, j % NQ])

  def gather_wait(blk, par):
    for j in range(GA):
      pltpu.make_async_copy(src_tab(j).at[pl.ds(0, 1)],
                            rows.at[par].at[pl.ds(j, 1)],
                            sem.at[par, j % NQ]).wait()

  gather_start(0, 0)

  def half(g, _):
    for par in range(2):
      blk = g * 2 + par
      nxt = blk + 1

      @pl.when(nxt < NBLK)
      def _():
        gather_start(nxt, (par + 1) % 2)

      gather_wait(blk, par)

      for i in range(IB):
        a = i * KA
        vc = [rows[par, a, pl.ds(c * L, L)] for c in range(DC)]

        def row_fma(r, acc4):
          return [acc4[c] + rows[par, r, pl.ds(c * L, L)] * vc[c]
                  for c in range(DC)]

        accp4 = row_fma(a + 1, [jnp.zeros((L,), jnp.float32)] * DC)
        accn4 = [jnp.zeros((L,), jnp.float32)] * DC
        for k in range(K):
          accn4 = row_fma(a + 2 + k, accn4)
        ooff = pl.multiple_of((blk * IB + i) * L, 8)
        accp_v[pl.ds(ooff, L)] = (accp4[0] + accp4[1]) + (accp4[2] + accp4[3])
        accn_v[pl.ds(ooff, L)] = (accn4[0] + accn4[1]) + (accn4[2] + accn4[3])
    return 0

  lax.fori_loop(0, NBLK // 2, half, 0)

  obase = pl.multiple_of(w * ipw * L, 8)
  pltpu.sync_copy(accp_v, accp_hbm.at[pl.ds(obase, ipw * L)])
  pltpu.sync_copy(accn_v, accn_hbm.at[pl.ds(obase, ipw * L)])


def _tc_finish(nitems, accp_ref, accn_ref, out_ref):
  rows, lanes = accp_ref.shape
  g = lanes // L
  sel = (lax.broadcasted_iota(jnp.int32, (lanes, g), 0) // L ==
         lax.broadcasted_iota(jnp.int32, (lanes, g), 1)).astype(jnp.float32)
  pos = jnp.dot(accp_ref[...], sel, preferred_element_type=jnp.float32)
  negdot = jnp.dot(accn_ref[...], sel, preferred_element_type=jnp.float32)
  loss = jax.nn.log_sigmoid(pos) + jax.nn.log_sigmoid(-negdot)
  out_ref[0, 0] = -jnp.sum(loss) / nitems


def kernel(V, U, center_words, target_words, neg_words):
  B, K = neg_words.shape
  KA = K + 2
  GA = KA * IB
  ipw = B // NW
  NBLK = ipw // IB

  gidx = jnp.concatenate([center_words, target_words, neg_words], axis=1)
  gidx = gidx.astype(jnp.int32).reshape(NW, NBLK, GA)
  gidx = jnp.pad(gidx, ((0, 0), (0, 0), (0, G_PAD - GA)))

  sc = pl.kernel(
      functools.partial(_sc_body, K, NBLK),
      out_type=(jax.ShapeDtypeStruct((B * L,), jnp.float32),
                jax.ShapeDtypeStruct((B * L,), jnp.float32)),
      mesh=plsc.VectorSubcoreMesh(core_axis_name="c", subcore_axis_name="s"),
      compiler_params=pltpu.CompilerParams(use_tc_tiling_on_sc=True),
      scratch_types=[
          pltpu.VMEM((NBLK * G_PAD,), jnp.int32),
          pltpu.VMEM((2, GA, D), jnp.float32),
          pltpu.VMEM((ipw * L,), jnp.float32),
          pltpu.VMEM((ipw * L,), jnp.float32),
          pltpu.SemaphoreType.DMA((2, 4)),
      ],
  )
  accp, accn = sc(V, U, gidx.reshape(NW, NBLK * G_PAD))

  out = pl.pallas_call(
      functools.partial(_tc_finish, float(B)),
      out_shape=jax.ShapeDtypeStruct((1, 1), jnp.float32),
      out_specs=pl.BlockSpec(memory_space=pltpu.SMEM),
  )(accp.reshape(B * L // 128, 128), accn.reshape(B * L // 128, 128))
  return out.reshape(())
